# Initial kernel scaffold; baseline (speedup 1.0000x reference)
import jax
import jax.numpy as jnp
from jax.experimental import pallas as pl


def kernel(x, edges, ew1, eb1, ew2, eb2, nw1, nb1, nw2, nb2, dw, db):
    raise NotImplementedError("write your pallas kernel here")


# speedup vs baseline: 74.3402x; 74.3402x over previous
import functools

import jax
import jax.numpy as jnp
from jax import lax
from jax.experimental import pallas as pl
from jax.experimental.pallas import tpu as pltpu
from jax.experimental.pallas import tpu_sc as plsc

N_NODES = 10000
N_EDGES = 160000
IN_DIM = 128
HID = 16
OUT = 8
BATCH = 4

NW = 32
CH = 128
NCHUNKS = N_EDGES // CH
CH_PER_W = -(-NCHUNKS // NW)
N_PAD = 10240
SLAB = N_PAD // 16
TN = 2000
F32 = jnp.float32


def _tc_project_body(x_ref, wa_ref, wb_ref, wn1_ref, eb1_ref, nb1_ref,
                     nw2_ref, nb2_ref, xa_ref, xb_ref, h_ref):
    for b in range(BATCH):
        xt = x_ref[b]
        va = jnp.dot(xt, wa_ref[...], preferred_element_type=F32) + eb1_ref[...]
        vb = jnp.dot(xt, wb_ref[...], preferred_element_type=F32)
        hm = jnp.maximum(
            jnp.dot(xt, wn1_ref[...], preferred_element_type=F32) + nb1_ref[...],
            0.0)
        hh = jnp.dot(hm, nw2_ref[...], preferred_element_type=F32) + nb2_ref[...]
        xa_ref[:, b, :] = va
        xb_ref[:, b, :] = vb
        h_ref[:, b, :] = hh


def _tc_project(x, wa, wb, wn1, eb1, nb1, nw2, nb2):
    grid = (N_NODES // TN,)
    full = lambda *shape: pl.BlockSpec(shape, lambda n: (0,) * len(shape))
    return pl.pallas_call(
        _tc_project_body,
        grid=grid,
        in_specs=[
            pl.BlockSpec((BATCH, TN, IN_DIM), lambda n: (0, n, 0)),
            full(IN_DIM, HID), full(IN_DIM, HID), full(IN_DIM, HID),
            full(1, HID), full(1, HID), full(HID, OUT), full(1, OUT),
        ],
        out_specs=[
            pl.BlockSpec((TN, BATCH, HID), lambda n: (n, 0, 0)),
            pl.BlockSpec((TN, BATCH, HID), lambda n: (n, 0, 0)),
            pl.BlockSpec((TN, BATCH, OUT), lambda n: (n, 0, 0)),
        ],
        out_shape=[
            jax.ShapeDtypeStruct((N_NODES, BATCH, HID), F32),
            jax.ShapeDtypeStruct((N_NODES, BATCH, HID), F32),
            jax.ShapeDtypeStruct((N_NODES, BATCH, OUT), F32),
        ],
    )(x, wa, wb, wn1, eb1, nb1, nw2, nb2)


WROW = BATCH * HID + 16


def _sc_edge_body(xa_hbm, xb_hbm, u_hbm, v_hbm, z_hbm,
                  msg_out,
                  idxu, idxv, arows, brows, rbuf, acc, sema, semb):
    cid = lax.axis_index("c")
    sid = lax.axis_index("s")
    wid = cid * 16 + sid
    base_rows = sid * SLAB

    pltpu.sync_copy(z_hbm, acc.at[pl.ds(base_rows, SLAB), :])

    def fill_body(i, c):
        rbuf[i, pl.ds(BATCH * HID, 16)] = jnp.full((16,), 1.0, F32)
        return c

    lax.fori_loop(0, CH, fill_body, 0)
    plsc.subcore_barrier()

    def chunk_body(j, carry):
        g = j * NW + wid

        @pl.when(g < NCHUNKS)
        def _():
            ebase = g * CH
            pltpu.sync_copy(u_hbm.at[pl.ds(ebase, CH)], idxu.at[0])
            pltpu.sync_copy(v_hbm.at[pl.ds(ebase, CH)], idxv.at[0])
            ga = pltpu.async_copy(xa_hbm.at[idxu.at[0]], arows, sema)
            gb = pltpu.async_copy(xb_hbm.at[idxv.at[0]], brows, semb)
            ga.wait()
            gb.wait()

            def row_body(i, c):
                for k in range(BATCH):
                    s = k * HID
                    rbuf[i, pl.ds(s, HID)] = jnp.maximum(
                        arows[i, pl.ds(s, HID)] + brows[i, pl.ds(s, HID)], 0.0)
                return c

            lax.fori_loop(0, CH, row_body, 0)

            pltpu.sync_copy(rbuf, acc.at[idxu.at[0]], add=True)
            pltpu.sync_copy(rbuf, acc.at[idxv.at[0]], add=True)

        return carry

    lax.fori_loop(0, CH_PER_W, chunk_body, 0)
    plsc.subcore_barrier()

    pltpu.sync_copy(acc.at[pl.ds(base_rows, SLAB), :],
                    msg_out.at[cid, pl.ds(base_rows, SLAB), :])


@functools.cache
def _sc_edge():
    return functools.partial(
        pl.kernel,
        out_type=jax.ShapeDtypeStruct((2, N_PAD, WROW), F32),
        mesh=plsc.VectorSubcoreMesh(core_axis_name="c", subcore_axis_name="s",
                                    num_cores=2, num_subcores=16),
        compiler_params=pltpu.CompilerParams(use_tc_tiling_on_sc=False),
        scratch_types=[
            pltpu.VMEM((1, CH), jnp.int32),
            pltpu.VMEM((1, CH), jnp.int32),
            pltpu.VMEM((CH, BATCH * HID), F32),
            pltpu.VMEM((CH, BATCH * HID), F32),
            pltpu.VMEM((CH, WROW), F32),
            pltpu.VMEM_SHARED((N_PAD, WROW), F32),
            pltpu.SemaphoreType.DMA,
            pltpu.SemaphoreType.DMA,
        ],
    )(_sc_edge_body)


def _tc_recon_body(h_ref, msg_ref, ew2_ref, eb2_ref, dw_ref, db_ref,
                   out_ref):
    m2 = jnp.dot(ew2_ref[...], dw_ref[...], preferred_element_type=F32)
    c2 = jnp.dot(eb2_ref[...], dw_ref[...], preferred_element_type=F32)
    dg16 = msg_ref[0, :, BATCH, :] + msg_ref[1, :, BATCH, :]
    dg = dg16[:, 0:1]
    degterm = jnp.dot(dg, c2, preferred_element_type=F32)
    for b in range(BATCH):
        mb = msg_ref[0, :, b, :] + msg_ref[1, :, b, :]
        hb = h_ref[:, b, :]
        ob = (jnp.dot(hb, dw_ref[...], preferred_element_type=F32)
              + jnp.dot(mb, m2, preferred_element_type=F32)
              + degterm + db_ref[...])
        out_ref[b] = ob


def _tc_recon(h, msg, ew2, eb2, dw, db):
    grid = (N_NODES // TN,)
    full = lambda *shape: pl.BlockSpec(shape, lambda n: (0,) * len(shape))
    return pl.pallas_call(
        _tc_recon_body,
        grid=grid,
        in_specs=[
            pl.BlockSpec((TN, BATCH, OUT), lambda n: (n, 0, 0)),
            pl.BlockSpec((2, TN, BATCH + 1, HID), lambda n: (0, n, 0, 0)),
            full(HID, OUT), full(1, OUT), full(OUT, IN_DIM), full(1, IN_DIM),
        ],
        out_specs=pl.BlockSpec((BATCH, TN, IN_DIM), lambda n: (0, n, 0)),
        out_shape=jax.ShapeDtypeStruct((BATCH, N_NODES, IN_DIM), F32),
    )(h, msg, ew2, eb2, dw, db)


def kernel(x, edges, ew1, eb1, ew2, eb2, nw1, nb1, nw2, nb2, dw, db):
    edges = edges.astype(jnp.int32)
    u = edges[:, 0]
    v = edges[:, 1]
    wa = ew1[:IN_DIM]
    wb = ew1[IN_DIM:]

    xa, xb, h = _tc_project(
        x, wa, wb, nw1,
        eb1.reshape(1, HID), nb1.reshape(1, HID), nw2, nb2.reshape(1, OUT))

    xa2 = xa.reshape(N_NODES, BATCH * HID)
    xb2 = xb.reshape(N_NODES, BATCH * HID)
    z = jnp.zeros((SLAB, WROW), F32)

    msg = _sc_edge()(xa2, xb2, u, v, z)

    msg4 = msg.reshape(2, N_PAD, BATCH + 1, HID)
    recon = _tc_recon(h, msg4, ew2, eb2.reshape(1, OUT), dw,
                      db.reshape(1, IN_DIM))
    return recon


# speedup vs baseline: 95.6374x; 1.2865x over previous
import functools

import jax
import jax.numpy as jnp
from jax import lax
from jax.experimental import pallas as pl
from jax.experimental.pallas import tpu as pltpu
from jax.experimental.pallas import tpu_sc as plsc

N_NODES = 10000
N_EDGES = 160000
IN_DIM = 128
HID = 16
OUT = 8
BATCH = 4

NW = 32
EPT = N_EDGES // NW
CHW = 125
CHP = 128
NJ = EPT // CHW
DUMMY = 10239
N_PAD = 10240
SLAB = N_PAD // 16
TN = 2000
F32 = jnp.float32


def _tc_project_body(x_ref, wa_ref, wb_ref, wn1_ref, eb1_ref, nb1_ref,
                     nw2_ref, nb2_ref, xa_ref, xb_ref, h_ref):
    for b in range(BATCH):
        xt = x_ref[b]
        va = jnp.dot(xt, wa_ref[...], preferred_element_type=F32) + eb1_ref[...]
        vb = jnp.dot(xt, wb_ref[...], preferred_element_type=F32)
        hm = jnp.maximum(
            jnp.dot(xt, wn1_ref[...], preferred_element_type=F32) + nb1_ref[...],
            0.0)
        hh = jnp.dot(hm, nw2_ref[...], preferred_element_type=F32) + nb2_ref[...]
        xa_ref[:, b, :] = va
        xb_ref[:, b, :] = vb
        h_ref[:, b, :] = hh


def _tc_project(x, wa, wb, wn1, eb1, nb1, nw2, nb2):
    grid = (N_NODES // TN,)
    full = lambda *shape: pl.BlockSpec(shape, lambda n: (0,) * len(shape))
    return pl.pallas_call(
        _tc_project_body,
        grid=grid,
        in_specs=[
            pl.BlockSpec((BATCH, TN, IN_DIM), lambda n: (0, n, 0)),
            full(IN_DIM, HID), full(IN_DIM, HID), full(IN_DIM, HID),
            full(1, HID), full(1, HID), full(HID, OUT), full(1, OUT),
        ],
        out_specs=[
            pl.BlockSpec((TN, BATCH, HID), lambda n: (n, 0, 0)),
            pl.BlockSpec((TN, BATCH, HID), lambda n: (n, 0, 0)),
            pl.BlockSpec((TN, BATCH, OUT), lambda n: (n, 0, 0)),
        ],
        out_shape=[
            jax.ShapeDtypeStruct((N_PAD, BATCH, HID), F32),
            jax.ShapeDtypeStruct((N_PAD, BATCH, HID), F32),
            jax.ShapeDtypeStruct((N_PAD, BATCH, OUT), F32),
        ],
    )(x, wa, wb, wn1, eb1, nb1, nw2, nb2)


WROW = BATCH * HID + 16


def _sc_edge_body(xa_hbm, xb_hbm, u_hbm, v_hbm, z_hbm,
                  msg_out,
                  idxu, idxv, a0, a1, b0, b1, r0, r1, acc,
                  sga0, sga1, sgb0, sgb1, ssu0, ssu1, ssv0, ssv1):
    cid = lax.axis_index("c")
    sid = lax.axis_index("s")
    wid = cid * 16 + sid
    base_rows = sid * SLAB

    gbufs = ((a0, b0, sga0, sgb0), (a1, b1, sga1, sgb1))
    sbufs = ((r0, ssu0, ssv0), (r1, ssu1, ssv1))

    pltpu.sync_copy(z_hbm, acc.at[pl.ds(base_rows, SLAB), :])

    def fill_body(i, c):
        r0[i, pl.ds(BATCH * HID, 16)] = jnp.full((16,), 1.0, F32)
        r1[i, pl.ds(BATCH * HID, 16)] = jnp.full((16,), 1.0, F32)
        return c

    lax.fori_loop(0, CHP, fill_body, 0)

    pltpu.sync_copy(u_hbm.at[wid], idxu)
    pltpu.sync_copy(v_hbm.at[wid], idxv)
    plsc.subcore_barrier()

    def issue_gather(j, slot):
        a, b, sa, sb = gbufs[slot]
        pltpu.async_copy(xa_hbm.at[idxu.at[j]], a, sa)
        pltpu.async_copy(xb_hbm.at[idxv.at[j]], b, sb)

    def wait_gather(j, slot):
        a, b, sa, sb = gbufs[slot]
        pltpu.make_async_copy(xa_hbm.at[idxu.at[j]], a, sa).wait()
        pltpu.make_async_copy(xb_hbm.at[idxv.at[j]], b, sb).wait()

    def issue_scatter(j, slot):
        r, su, sv = sbufs[slot]
        pltpu.async_copy(r, acc.at[idxu.at[j]], su, add=True)
        pltpu.async_copy(r, acc.at[idxv.at[j]], sv, add=True)

    def wait_scatter(j, slot):
        r, su, sv = sbufs[slot]
        pltpu.make_async_copy(r, acc.at[idxu.at[j]], su).wait()
        pltpu.make_async_copy(r, acc.at[idxv.at[j]], sv).wait()

    def compute(slot):
        a, b, _, _ = gbufs[slot]
        r, _, _ = sbufs[slot]

        def row_body(i, c):
            for k in range(BATCH):
                s = k * HID
                r[i, pl.ds(s, HID)] = jnp.maximum(
                    a[i, pl.ds(s, HID)] + b[i, pl.ds(s, HID)], 0.0)
            return c

        lax.fori_loop(0, CHP, row_body, 0)

    issue_gather(0, 0)

    def j2_body(j2, carry):
        for phase in range(2):
            j = j2 * 2 + phase
            wait_gather(j, phase)

            @pl.when(j + 1 < NJ)
            def _():
                issue_gather(j + 1, 1 - phase)

            @pl.when(j >= 2)
            def _():
                wait_scatter(j - 2, phase)

            compute(phase)
            issue_scatter(j, phase)
        return carry

    lax.fori_loop(0, NJ // 2, j2_body, 0)
    wait_scatter(NJ - 2, 0)
    wait_scatter(NJ - 1, 1)
    plsc.subcore_barrier()

    pltpu.sync_copy(acc.at[pl.ds(base_rows, SLAB), :],
                    msg_out.at[cid, pl.ds(base_rows, SLAB), :])


@functools.cache
def _sc_edge():
    return functools.partial(
        pl.kernel,
        out_type=jax.ShapeDtypeStruct((2, N_PAD, WROW), F32),
        mesh=plsc.VectorSubcoreMesh(core_axis_name="c", subcore_axis_name="s",
                                    num_cores=2, num_subcores=16),
        compiler_params=pltpu.CompilerParams(use_tc_tiling_on_sc=False),
        scratch_types=[
            pltpu.VMEM((NJ, CHP), jnp.int32),
            pltpu.VMEM((NJ, CHP), jnp.int32),
            pltpu.VMEM((CHP, BATCH * HID), F32),
            pltpu.VMEM((CHP, BATCH * HID), F32),
            pltpu.VMEM((CHP, BATCH * HID), F32),
            pltpu.VMEM((CHP, BATCH * HID), F32),
            pltpu.VMEM((CHP, WROW), F32),
            pltpu.VMEM((CHP, WROW), F32),
            pltpu.VMEM_SHARED((N_PAD, WROW), F32),
            pltpu.SemaphoreType.DMA,
            pltpu.SemaphoreType.DMA,
            pltpu.SemaphoreType.DMA,
            pltpu.SemaphoreType.DMA,
            pltpu.SemaphoreType.DMA,
            pltpu.SemaphoreType.DMA,
            pltpu.SemaphoreType.DMA,
            pltpu.SemaphoreType.DMA,
        ],
    )(_sc_edge_body)


def _tc_recon_body(h_ref, msg_ref, ew2_ref, eb2_ref, dw_ref, db_ref,
                   out_ref):
    m2 = jnp.dot(ew2_ref[...], dw_ref[...], preferred_element_type=F32)
    c2 = jnp.dot(eb2_ref[...], dw_ref[...], preferred_element_type=F32)
    dg16 = msg_ref[0, :, BATCH, :] + msg_ref[1, :, BATCH, :]
    dg = dg16[:, 0:1]
    degterm = jnp.dot(dg, c2, preferred_element_type=F32)
    for b in range(BATCH):
        mb = msg_ref[0, :, b, :] + msg_ref[1, :, b, :]
        hb = h_ref[:, b, :]
        ob = (jnp.dot(hb, dw_ref[...], preferred_element_type=F32)
              + jnp.dot(mb, m2, preferred_element_type=F32)
              + degterm + db_ref[...])
        out_ref[b] = ob


def _tc_recon(h, msg, ew2, eb2, dw, db):
    grid = (N_NODES // TN,)
    full = lambda *shape: pl.BlockSpec(shape, lambda n: (0,) * len(shape))
    return pl.pallas_call(
        _tc_recon_body,
        grid=grid,
        in_specs=[
            pl.BlockSpec((TN, BATCH, OUT), lambda n: (n, 0, 0)),
            pl.BlockSpec((2, TN, BATCH + 1, HID), lambda n: (0, n, 0, 0)),
            full(HID, OUT), full(1, OUT), full(OUT, IN_DIM), full(1, IN_DIM),
        ],
        out_specs=pl.BlockSpec((BATCH, TN, IN_DIM), lambda n: (0, n, 0)),
        out_shape=jax.ShapeDtypeStruct((BATCH, N_NODES, IN_DIM), F32),
    )(h, msg, ew2, eb2, dw, db)


def kernel(x, edges, ew1, eb1, ew2, eb2, nw1, nb1, nw2, nb2, dw, db):
    edges = edges.astype(jnp.int32)
    u = edges[:, 0]
    v = edges[:, 1]
    wa = ew1[:IN_DIM]
    wb = ew1[IN_DIM:]

    xa, xb, h = _tc_project(
        x, wa, wb, nw1,
        eb1.reshape(1, HID), nb1.reshape(1, HID), nw2, nb2.reshape(1, OUT))

    xa2 = xa.reshape(N_PAD, BATCH * HID)
    xb2 = xb.reshape(N_PAD, BATCH * HID)
    pad = jnp.full((NW, NJ, CHP - CHW), DUMMY, jnp.int32)
    u3 = jnp.concatenate([u.reshape(NW, NJ, CHW), pad], axis=-1)
    v3 = jnp.concatenate([v.reshape(NW, NJ, CHW), pad], axis=-1)
    z = jnp.zeros((SLAB, WROW), F32)

    msg = _sc_edge()(xa2, xb2, u3, v3, z)

    msg4 = msg.reshape(2, N_PAD, BATCH + 1, HID)
    recon = _tc_recon(h, msg4, ew2, eb2.reshape(1, OUT), dw,
                      db.reshape(1, IN_DIM))
    return recon


# speedup vs baseline: 157.3250x; 1.6450x over previous
import functools

import jax
import jax.numpy as jnp
from jax import lax
from jax.experimental import pallas as pl
from jax.experimental import layout as jex_layout
from jax.experimental.pallas import tpu as pltpu
from jax.experimental.pallas import tpu_sc as plsc

N_NODES = 10000
N_EDGES = 160000
IN_DIM = 128
HID = 16
OUT = 8
BATCH = 4

NW = 32
EPT = N_EDGES // NW
CHW = 125
CHP = 128
NJ = EPT // CHW
DUMMY = 10239
N_PAD = 10240
SLAB = N_PAD // 16
TN = 2000
F32 = jnp.float32


def _tc_project_body(x_ref, wa_ref, wb_ref, wn1_ref, eb1_ref, nb1_ref,
                     nw2_ref, nb2_ref, xa_ref, xb_ref, h_ref):
    vas, vbs, hhs = [], [], []
    for b in range(BATCH):
        xt = x_ref[b]
        vas.append(
            jnp.dot(xt, wa_ref[...], preferred_element_type=F32) + eb1_ref[...])
        vbs.append(jnp.dot(xt, wb_ref[...], preferred_element_type=F32))
        hm = jnp.maximum(
            jnp.dot(xt, wn1_ref[...], preferred_element_type=F32) + nb1_ref[...],
            0.0)
        hhs.append(
            jnp.dot(hm, nw2_ref[...], preferred_element_type=F32) + nb2_ref[...])
    xa_ref[...] = jnp.concatenate(vas, axis=1)
    xb_ref[...] = jnp.concatenate(vbs, axis=1)
    h_ref[...] = jnp.concatenate(hhs, axis=1)


def _tc_project(x, wa, wb, wn1, eb1, nb1, nw2, nb2):
    grid = (N_NODES // TN,)
    full = lambda *shape: pl.BlockSpec(shape, lambda n: (0,) * len(shape))
    return pl.pallas_call(
        _tc_project_body,
        grid=grid,
        in_specs=[
            pl.BlockSpec((BATCH, TN, IN_DIM), lambda n: (0, n, 0)),
            full(IN_DIM, HID), full(IN_DIM, HID), full(IN_DIM, HID),
            full(1, HID), full(1, HID), full(HID, OUT), full(1, OUT),
        ],
        out_specs=[
            pl.BlockSpec((TN, BATCH * HID), lambda n: (n, 0)),
            pl.BlockSpec((TN, BATCH * HID), lambda n: (n, 0)),
            pl.BlockSpec((TN, BATCH * OUT), lambda n: (n, 0)),
        ],
        out_shape=[
            jax.ShapeDtypeStruct((N_PAD, BATCH * HID), F32),
            jax.ShapeDtypeStruct((N_PAD, BATCH * HID), F32),
            jax.ShapeDtypeStruct((N_PAD, BATCH * OUT), F32),
        ],
    )(x, wa, wb, wn1, eb1, nb1, nw2, nb2)


WROW = BATCH * HID + 16


def _sc_edge_body(xa_hbm, xb_hbm, u_hbm, v_hbm, z_hbm,
                  msg_out,
                  idxu, idxv, a0, a1, b0, b1, r0, r1, acc,
                  sga0, sga1, sgb0, sgb1, ssu0, ssu1, ssv0, ssv1):
    cid = lax.axis_index("c")
    sid = lax.axis_index("s")
    wid = cid * 16 + sid
    base_rows = sid * SLAB

    gbufs = ((a0, b0, sga0, sgb0), (a1, b1, sga1, sgb1))
    sbufs = ((r0, ssu0, ssv0), (r1, ssu1, ssv1))

    pltpu.sync_copy(z_hbm, acc.at[pl.ds(base_rows, SLAB), :])

    def fill_body(i, c):
        r0[i, pl.ds(BATCH * HID, 16)] = jnp.full((16,), 1.0, F32)
        r1[i, pl.ds(BATCH * HID, 16)] = jnp.full((16,), 1.0, F32)
        return c

    lax.fori_loop(0, CHP, fill_body, 0)

    pltpu.sync_copy(u_hbm.at[wid], idxu)
    pltpu.sync_copy(v_hbm.at[wid], idxv)
    plsc.subcore_barrier()

    def issue_gather(j, slot):
        a, b, sa, sb = gbufs[slot]
        pltpu.async_copy(xa_hbm.at[idxu.at[j]], a, sa)
        pltpu.async_copy(xb_hbm.at[idxv.at[j]], b, sb)

    def wait_gather(j, slot):
        a, b, sa, sb = gbufs[slot]
        pltpu.make_async_copy(xa_hbm.at[idxu.at[j]], a, sa).wait()
        pltpu.make_async_copy(xb_hbm.at[idxv.at[j]], b, sb).wait()

    def issue_scatter(j, slot):
        r, su, sv = sbufs[slot]
        pltpu.async_copy(r, acc.at[idxu.at[j]], su, add=True)
        pltpu.async_copy(r, acc.at[idxv.at[j]], sv, add=True)

    def wait_scatter(j, slot):
        r, su, sv = sbufs[slot]
        pltpu.make_async_copy(r, acc.at[idxu.at[j]], su).wait()
        pltpu.make_async_copy(r, acc.at[idxv.at[j]], sv).wait()

    def compute(slot):
        a, b, _, _ = gbufs[slot]
        r, _, _ = sbufs[slot]

        def row_body(i, c):
            for k in range(BATCH):
                s = k * HID
                r[i, pl.ds(s, HID)] = jnp.maximum(
                    a[i, pl.ds(s, HID)] + b[i, pl.ds(s, HID)], 0.0)
            return c

        lax.fori_loop(0, CHP, row_body, 0)

    issue_gather(0, 0)

    def j2_body(j2, carry):
        for phase in range(2):
            j = j2 * 2 + phase
            wait_gather(j, phase)

            @pl.when(j + 1 < NJ)
            def _():
                issue_gather(j + 1, 1 - phase)

            @pl.when(j >= 2)
            def _():
                wait_scatter(j - 2, phase)

            compute(phase)
            issue_scatter(j, phase)
        return carry

    lax.fori_loop(0, NJ // 2, j2_body, 0)
    wait_scatter(NJ - 2, 0)
    wait_scatter(NJ - 1, 1)
    plsc.subcore_barrier()

    pltpu.sync_copy(acc.at[pl.ds(base_rows, SLAB), :],
                    msg_out.at[cid, pl.ds(base_rows, SLAB), :])


@functools.cache
def _sc_edge():
    return functools.partial(
        pl.kernel,
        out_type=jax.ShapeDtypeStruct((2, N_PAD, WROW), F32),
        mesh=plsc.VectorSubcoreMesh(core_axis_name="c", subcore_axis_name="s",
                                    num_cores=2, num_subcores=16),
        compiler_params=pltpu.CompilerParams(use_tc_tiling_on_sc=False),
        scratch_types=[
            pltpu.VMEM((NJ, CHP), jnp.int32),
            pltpu.VMEM((NJ, CHP), jnp.int32),
            pltpu.VMEM((CHP, BATCH * HID), F32),
            pltpu.VMEM((CHP, BATCH * HID), F32),
            pltpu.VMEM((CHP, BATCH * HID), F32),
            pltpu.VMEM((CHP, BATCH * HID), F32),
            pltpu.VMEM((CHP, WROW), F32),
            pltpu.VMEM((CHP, WROW), F32),
            pltpu.VMEM_SHARED((N_PAD, WROW), F32),
            pltpu.SemaphoreType.DMA,
            pltpu.SemaphoreType.DMA,
            pltpu.SemaphoreType.DMA,
            pltpu.SemaphoreType.DMA,
            pltpu.SemaphoreType.DMA,
            pltpu.SemaphoreType.DMA,
            pltpu.SemaphoreType.DMA,
            pltpu.SemaphoreType.DMA,
        ],
    )(_sc_edge_body)


def _tc_recon_body(h_ref, msg_ref, ew2_ref, eb2_ref, dw_ref, db_ref,
                   out_ref):
    m2 = jnp.dot(ew2_ref[...], dw_ref[...], preferred_element_type=F32)
    c2 = jnp.dot(eb2_ref[...], dw_ref[...], preferred_element_type=F32)
    m = msg_ref[0] + msg_ref[1]
    dg = m[:, BATCH * HID:BATCH * HID + 1]
    degterm = jnp.dot(dg, c2, preferred_element_type=F32)
    for b in range(BATCH):
        mb = m[:, b * HID:(b + 1) * HID]
        hb = h_ref[:, b * OUT:(b + 1) * OUT]
        ob = (jnp.dot(hb, dw_ref[...], preferred_element_type=F32)
              + jnp.dot(mb, m2, preferred_element_type=F32)
              + degterm + db_ref[...])
        out_ref[b] = ob


def _tc_recon(h, msg, ew2, eb2, dw, db):
    grid = (N_NODES // TN,)
    full = lambda *shape: pl.BlockSpec(shape, lambda n: (0,) * len(shape))
    return pl.pallas_call(
        _tc_recon_body,
        grid=grid,
        in_specs=[
            pl.BlockSpec((TN, BATCH * OUT), lambda n: (n, 0)),
            pl.BlockSpec((2, TN, WROW), lambda n: (0, n, 0)),
            full(HID, OUT), full(1, OUT), full(OUT, IN_DIM), full(1, IN_DIM),
        ],
        out_specs=pl.BlockSpec((BATCH, TN, IN_DIM), lambda n: (0, n, 0)),
        out_shape=jax.ShapeDtypeStruct((BATCH, N_NODES, IN_DIM), F32),
    )(h, msg, ew2, eb2, dw, db)


def kernel(x, edges, ew1, eb1, ew2, eb2, nw1, nb1, nw2, nb2, dw, db):
    edges = edges.astype(jnp.int32)
    u = edges[:, 0]
    v = edges[:, 1]
    wa = ew1[:IN_DIM]
    wb = ew1[IN_DIM:]

    xa, xb, h = _tc_project(
        x, wa, wb, nw1,
        eb1.reshape(1, HID), nb1.reshape(1, HID), nw2, nb2.reshape(1, OUT))

    pad = jnp.full((NW, NJ, CHP - CHW), DUMMY, jnp.int32)
    u3 = jnp.concatenate([u.reshape(NW, NJ, CHW), pad], axis=-1)
    v3 = jnp.concatenate([v.reshape(NW, NJ, CHW), pad], axis=-1)
    z = jnp.zeros((SLAB, WROW), F32)

    msg = _sc_edge()(xa, xb, u3, v3, z)

    recon = _tc_recon(h, msg, ew2, eb2.reshape(1, OUT), dw,
                      db.reshape(1, IN_DIM))
    return recon


# speedup vs baseline: 157.4414x; 1.0007x over previous
import functools

import jax
import jax.numpy as jnp
from jax import lax
from jax.experimental import pallas as pl
from jax.experimental.pallas import tpu as pltpu
from jax.experimental.pallas import tpu_sc as plsc

N_NODES = 10000
N_EDGES = 160000
IN_DIM = 128
HID = 16
OUT = 8
BATCH = 4

NW = 32
EPT = N_EDGES // NW
CHW = 125
CHP = 128
NJ = EPT // CHW
DUMMY = 10239
N_PAD = 10240
SLAB = N_PAD // 16
TN = 2000
F32 = jnp.float32


def _tc_project_body(x_ref, wa_ref, wb_ref, wn1_ref, eb1_ref, nb1_ref,
                     nw2_ref, nb2_ref, xa_ref, xb_ref, h_ref):
    vas, vbs, hhs = [], [], []
    for b in range(BATCH):
        xt = x_ref[b]
        vas.append(
            jnp.dot(xt, wa_ref[...], preferred_element_type=F32) + eb1_ref[...])
        vbs.append(jnp.dot(xt, wb_ref[...], preferred_element_type=F32))
        hm = jnp.maximum(
            jnp.dot(xt, wn1_ref[...], preferred_element_type=F32) + nb1_ref[...],
            0.0)
        hhs.append(
            jnp.dot(hm, nw2_ref[...], preferred_element_type=F32) + nb2_ref[...])
    xa_ref[...] = jnp.concatenate(vas, axis=1)
    xb_ref[...] = jnp.concatenate(vbs, axis=1)
    h_ref[...] = jnp.concatenate(hhs, axis=1)


def _tc_project(x, wa, wb, wn1, eb1, nb1, nw2, nb2):
    grid = (N_NODES // TN,)
    full = lambda *shape: pl.BlockSpec(shape, lambda n: (0,) * len(shape))
    return pl.pallas_call(
        _tc_project_body,
        grid=grid,
        in_specs=[
            pl.BlockSpec((BATCH, TN, IN_DIM), lambda n: (0, n, 0)),
            full(IN_DIM, HID), full(IN_DIM, HID), full(IN_DIM, HID),
            full(1, HID), full(1, HID), full(HID, OUT), full(1, OUT),
        ],
        out_specs=[
            pl.BlockSpec((TN, BATCH * HID), lambda n: (n, 0)),
            pl.BlockSpec((TN, BATCH * HID), lambda n: (n, 0)),
            pl.BlockSpec((TN, BATCH * OUT), lambda n: (n, 0)),
        ],
        out_shape=[
            jax.ShapeDtypeStruct((N_PAD, BATCH * HID), F32),
            jax.ShapeDtypeStruct((N_PAD, BATCH * HID), F32),
            jax.ShapeDtypeStruct((N_PAD, BATCH * OUT), F32),
        ],
    )(x, wa, wb, wn1, eb1, nb1, nw2, nb2)


WROW = BATCH * HID + 16


def _sc_edge_body(xa_hbm, xb_hbm, u_hbm, v_hbm, z_hbm,
                  msg_out,
                  idxu, idxv, a0, a1, b0, b1, r0, r1, acc,
                  sga0, sga1, sgb0, sgb1, ssu0, ssu1, ssv0, ssv1):
    cid = lax.axis_index("c")
    sid = lax.axis_index("s")
    wid = cid * 16 + sid
    base_rows = sid * SLAB

    gbufs = ((a0, b0, sga0, sgb0), (a1, b1, sga1, sgb1))
    sbufs = ((r0, ssu0, ssv0), (r1, ssu1, ssv1))

    pltpu.sync_copy(z_hbm, acc.at[pl.ds(base_rows, SLAB), :])

    def fill_body(i, c):
        r0[i, pl.ds(BATCH * HID, 16)] = jnp.full((16,), 1.0, F32)
        r1[i, pl.ds(BATCH * HID, 16)] = jnp.full((16,), 1.0, F32)
        return c

    lax.fori_loop(0, CHP, fill_body, 0)

    pltpu.sync_copy(u_hbm.at[wid], idxu)
    pltpu.sync_copy(v_hbm.at[wid], idxv)
    plsc.subcore_barrier()

    def issue_gather(j, slot):
        a, b, sa, sb = gbufs[slot]
        pltpu.async_copy(xa_hbm.at[idxu.at[j]], a, sa)
        pltpu.async_copy(xb_hbm.at[idxv.at[j]], b, sb)

    def wait_gather(j, slot):
        a, b, sa, sb = gbufs[slot]
        pltpu.make_async_copy(xa_hbm.at[idxu.at[j]], a, sa).wait()
        pltpu.make_async_copy(xb_hbm.at[idxv.at[j]], b, sb).wait()

    def issue_scatter(j, slot):
        r, su, sv = sbufs[slot]
        pltpu.async_copy(r, acc.at[idxu.at[j]], su, add=True)
        pltpu.async_copy(r, acc.at[idxv.at[j]], sv, add=True)

    def wait_scatter(j, slot):
        r, su, sv = sbufs[slot]
        pltpu.make_async_copy(r, acc.at[idxu.at[j]], su).wait()
        pltpu.make_async_copy(r, acc.at[idxv.at[j]], sv).wait()

    def compute(slot):
        a, b, _, _ = gbufs[slot]
        r, _, _ = sbufs[slot]

        def row_body(i, c):
            for k in range(BATCH):
                s = k * HID
                r[i, pl.ds(s, HID)] = jnp.maximum(
                    a[i, pl.ds(s, HID)] + b[i, pl.ds(s, HID)], 0.0)
            return c

        lax.fori_loop(0, CHP, row_body, 0)

    issue_gather(0, 0)

    def j2_body(j2, carry):
        for phase in range(2):
            j = j2 * 2 + phase
            wait_gather(j, phase)

            @pl.when(j + 1 < NJ)
            def _():
                issue_gather(j + 1, 1 - phase)

            @pl.when(j >= 2)
            def _():
                wait_scatter(j - 2, phase)

            compute(phase)
            issue_scatter(j, phase)
        return carry

    lax.fori_loop(0, NJ // 2, j2_body, 0)
    wait_scatter(NJ - 2, 0)
    wait_scatter(NJ - 1, 1)
    plsc.subcore_barrier()

    pltpu.sync_copy(acc.at[pl.ds(base_rows, SLAB), :],
                    msg_out.at[cid, pl.ds(base_rows, SLAB), :])


@functools.cache
def _sc_edge():
    return functools.partial(
        pl.kernel,
        out_type=jax.ShapeDtypeStruct((2, N_PAD, WROW), F32),
        mesh=plsc.VectorSubcoreMesh(core_axis_name="c", subcore_axis_name="s",
                                    num_cores=2, num_subcores=16),
        compiler_params=pltpu.CompilerParams(use_tc_tiling_on_sc=False),
        scratch_types=[
            pltpu.VMEM((NJ, CHP), jnp.int32),
            pltpu.VMEM((NJ, CHP), jnp.int32),
            pltpu.VMEM((CHP, BATCH * HID), F32),
            pltpu.VMEM((CHP, BATCH * HID), F32),
            pltpu.VMEM((CHP, BATCH * HID), F32),
            pltpu.VMEM((CHP, BATCH * HID), F32),
            pltpu.VMEM((CHP, WROW), F32),
            pltpu.VMEM((CHP, WROW), F32),
            pltpu.VMEM_SHARED((N_PAD, WROW), F32),
            pltpu.SemaphoreType.DMA,
            pltpu.SemaphoreType.DMA,
            pltpu.SemaphoreType.DMA,
            pltpu.SemaphoreType.DMA,
            pltpu.SemaphoreType.DMA,
            pltpu.SemaphoreType.DMA,
            pltpu.SemaphoreType.DMA,
            pltpu.SemaphoreType.DMA,
        ],
    )(_sc_edge_body)


def _tc_recon_body(h_ref, msg_ref, ew2_ref, eb2_ref, dw_ref, db_ref,
                   out_ref):
    m2 = jnp.dot(ew2_ref[...], dw_ref[...], preferred_element_type=F32)
    c2 = jnp.dot(eb2_ref[...], dw_ref[...], preferred_element_type=F32)
    m = msg_ref[0] + msg_ref[1]
    dg = m[:, BATCH * HID:BATCH * HID + 1]
    degterm = jnp.dot(dg, c2, preferred_element_type=F32)
    for b in range(BATCH):
        mb = m[:, b * HID:(b + 1) * HID]
        hb = h_ref[:, b * OUT:(b + 1) * OUT]
        ob = (jnp.dot(hb, dw_ref[...], preferred_element_type=F32)
              + jnp.dot(mb, m2, preferred_element_type=F32)
              + degterm + db_ref[...])
        out_ref[b] = ob


def _tc_recon(h, msg, ew2, eb2, dw, db):
    grid = (N_NODES // TN,)
    full = lambda *shape: pl.BlockSpec(shape, lambda n: (0,) * len(shape))
    return pl.pallas_call(
        _tc_recon_body,
        grid=grid,
        in_specs=[
            pl.BlockSpec((TN, BATCH * OUT), lambda n: (n, 0)),
            pl.BlockSpec((2, TN, WROW), lambda n: (0, n, 0)),
            full(HID, OUT), full(1, OUT), full(OUT, IN_DIM), full(1, IN_DIM),
        ],
        out_specs=pl.BlockSpec((BATCH, TN, IN_DIM), lambda n: (0, n, 0)),
        out_shape=jax.ShapeDtypeStruct((BATCH, N_NODES, IN_DIM), F32),
    )(h, msg, ew2, eb2, dw, db)


def kernel(x, edges, ew1, eb1, ew2, eb2, nw1, nb1, nw2, nb2, dw, db):
    edges = edges.astype(jnp.int32)
    u = edges[:, 0]
    v = edges[:, 1]
    wa = ew1[:IN_DIM]
    wb = ew1[IN_DIM:]

    xa, xb, h = _tc_project(
        x, wa, wb, nw1,
        eb1.reshape(1, HID), nb1.reshape(1, HID), nw2, nb2.reshape(1, OUT))

    pad = jnp.full((NW, NJ, CHP - CHW), DUMMY, jnp.int32)
    u3 = jnp.concatenate([u.reshape(NW, NJ, CHW), pad], axis=-1)
    v3 = jnp.concatenate([v.reshape(NW, NJ, CHW), pad], axis=-1)
    z = jnp.zeros((SLAB, WROW), F32)

    msg = _sc_edge()(xa, xb, u3, v3, z)

    recon = _tc_recon(h, msg, ew2, eb2.reshape(1, OUT), dw,
                      db.reshape(1, IN_DIM))
    return recon
